# baseline (speedup 1.0000x reference)
import jax
import jax.numpy as jnp
from jax import lax
from jax.experimental import pallas as pl
from jax.experimental.pallas import tpu as pltpu
from jax.experimental.pallas import tpu_sc as plsc

B, L, V, C, NCLS = 1024, 50, 1000000, 64, 1000
NCORES, NSUB = 2, 16
NW = NCORES * NSUB
BAGS = 2 * B
BAGS_PER_W = BAGS // NW
CHUNK_BAGS = 8
NCHUNK = BAGS_PER_W // CHUNK_BAGS
CHUNK_IDX = CHUNK_BAGS * L
NG_FULL = CHUNK_IDX // 128
NG_TAIL = CHUNK_IDX - NG_FULL * 128
IDX_ROWS = NG_FULL + 1
LANES = 16
NQ = C // LANES
WPAD = 64


def _sc_body(emb_h, idx_h, w_h, ho_h, c_h, idx_v, w_v, ho_v, rows_v, c_v, sem):
    wid = lax.axis_index("s") * NCORES + lax.axis_index("c")

    def chunk_body(ch, chunk_carry):
        pltpu.sync_copy(idx_h.at[wid, ch], idx_v)
        pltpu.sync_copy(w_h.at[wid, ch], w_v)
        pltpu.sync_copy(ho_h.at[wid, ch], ho_v)
        copies = []
        for j in range(NG_FULL):
            cp = pltpu.make_async_copy(
                emb_h.at[idx_v.at[j]], rows_v.at[pl.ds(j * 128, 128)], sem)
            cp.start()
            copies.append(cp)
        cp = pltpu.make_async_copy(
            emb_h.at[idx_v.at[NG_FULL, pl.ds(0, NG_TAIL)]],
            rows_v.at[pl.ds(NG_FULL * 128, NG_TAIL)], sem)
        cp.start()
        copies.append(cp)
        for cp in copies:
            cp.wait()

        def bag_body(bag, carry):
            base = bag * L
            wv = [w_v[bag, pl.ds(q * LANES, LANES)] for q in range(NQ)]
            hv = [ho_v[bag, pl.ds(q * LANES, LANES)] for q in range(NQ)]
            acc = [jnp.zeros((LANES,), jnp.float32) for _ in range(NQ)]
            for l in range(L):
                wgt = wv[l // LANES][l % LANES]
                hof = hv[l // LANES][l % LANES]
                for q in range(NQ):
                    acc[q] = acc[q] + wgt * rows_v[
                        base + l, pl.ds(hof + q * LANES, LANES)]
            for q in range(NQ):
                c_v[bag, pl.ds(q * LANES, LANES)] = jnp.where(
                    acc[q] > 0.0, 1.0, 0.0).astype(jnp.float32)
            return carry

        lax.fori_loop(0, CHUNK_BAGS, bag_body, 0)
        pltpu.sync_copy(
            c_v, c_h.at[pl.ds(wid * BAGS_PER_W + ch * CHUNK_BAGS, CHUNK_BAGS)])
        return chunk_carry

    lax.fori_loop(0, NCHUNK, chunk_body, 0)


def _sparse_stage(emb2, idx_pad, w_arr, ho_arr):
    mesh = plsc.VectorSubcoreMesh(core_axis_name="c", subcore_axis_name="s")
    fn = pl.kernel(
        _sc_body,
        out_type=jax.ShapeDtypeStruct((BAGS, C), jnp.float32),
        mesh=mesh,
        compiler_params=pltpu.CompilerParams(use_tc_tiling_on_sc=False),
        scratch_types=[
            pltpu.VMEM((IDX_ROWS, 128), jnp.int32),
            pltpu.VMEM((CHUNK_BAGS, WPAD), jnp.float32),
            pltpu.VMEM((CHUNK_BAGS, WPAD), jnp.int32),
            pltpu.VMEM((CHUNK_IDX, 2 * C), jnp.float32),
            pltpu.VMEM((CHUNK_BAGS, C), jnp.float32),
            pltpu.SemaphoreType.DMA,
        ],
    )
    return fn(emb2, idx_pad, w_arr, ho_arr)


def _dec_body(c_ref, wt_ref, b_ref, o_ref):
    o_ref[...] = jnp.dot(
        c_ref[...], wt_ref[...],
        preferred_element_type=jnp.float32,
        precision=lax.Precision.HIGHEST,
    ) + b_ref[...]


def _decoder_stage(c, wt, b2):
    bm = 256
    return pl.pallas_call(
        _dec_body,
        grid=(BAGS // bm,),
        in_specs=[
            pl.BlockSpec((bm, C), lambda i: (i, 0)),
            pl.BlockSpec((C, NCLS), lambda i: (0, 0)),
            pl.BlockSpec((1, NCLS), lambda i: (0, 0)),
        ],
        out_specs=pl.BlockSpec((bm, NCLS), lambda i: (i, 0)),
        out_shape=jax.ShapeDtypeStruct((BAGS, NCLS), jnp.float32),
    )(c, wt, b2)


def kernel(v, emb, W_dec, b_dec):
    keys = v[:, :, 0, :2]
    vals = v[:, :, 1, :2]
    idx = jnp.transpose(keys, (2, 0, 1)).reshape(BAGS, L).astype(jnp.int32)
    wts = jnp.transpose(vals, (2, 0, 1)).reshape(BAGS, L)

    emb2 = emb.reshape(V // 2, 2 * C)
    slot = idx >> 1
    hoff = (idx & 1) * C

    idx_pad = jnp.pad(
        slot.reshape(NW, NCHUNK, CHUNK_IDX),
        ((0, 0), (0, 0), (0, IDX_ROWS * 128 - CHUNK_IDX)),
    ).reshape(NW, NCHUNK, IDX_ROWS, 128)
    w_arr = jnp.pad(wts, ((0, 0), (0, WPAD - L))).reshape(
        NW, NCHUNK, CHUNK_BAGS, WPAD)
    ho_arr = jnp.pad(hoff, ((0, 0), (0, WPAD - L))).reshape(
        NW, NCHUNK, CHUNK_BAGS, WPAD)

    c = _sparse_stage(emb2, idx_pad, w_arr, ho_arr)
    out = _decoder_stage(c, W_dec.T, b_dec.reshape(1, NCLS))
    o1 = out[:B]
    o2 = out[B:]
    return (o1, o2, o2)


# speedup vs baseline: 1.6201x; 1.6201x over previous
import jax
import jax.numpy as jnp
from jax import lax
from jax.experimental import pallas as pl
from jax.experimental.pallas import tpu as pltpu
from jax.experimental.pallas import tpu_sc as plsc

B, L, V, C, NCLS = 1024, 50, 1000000, 64, 1000
NCORES, NSUB = 2, 16
NW = NCORES * NSUB
BAGS = 2 * B
BAGS_PER_W = BAGS // NW
CHUNK_BAGS = 8
NCHUNK = BAGS_PER_W // CHUNK_BAGS
CHUNK_IDX = CHUNK_BAGS * L
NGROUP = CHUNK_IDX // 16
LANES = 16
NQ = C // LANES
WPAD = 64


def _sc_body(emb_h, idx_h, w_h, c_h, idx_v, w_v, rows0, rows1, c_v, sem0,
             sem1):
    wid = lax.axis_index("s") * NCORES + lax.axis_index("c")
    rows = (rows0, rows1)
    sems = (sem0, sem1)

    def fire_chunk(ch, par):
        pltpu.sync_copy(idx_h.at[wid, ch], idx_v)
        rv, sem = rows[par], sems[par]

        def fire_group(g, carry):
            iv = idx_v[pl.ds(g * 16, 16)]
            for j in range(16):
                pltpu.make_async_copy(
                    emb_h.at[iv[j]], rv.at[g * 16 + j], sem).start()
            return carry

        lax.fori_loop(0, NGROUP, fire_group, 0)

    def drain_chunk(par):
        pltpu.make_async_copy(
            emb_h.at[pl.ds(0, CHUNK_IDX)], rows[par], sems[par]).wait()

    def compute_chunk(ch, par):
        pltpu.sync_copy(w_h.at[wid, ch], w_v)
        rv = rows[par]

        def bag_body(bag, carry):
            base = bag * L
            wv = [w_v[bag, pl.ds(q * LANES, LANES)] for q in range(NQ)]
            acc = [jnp.zeros((LANES,), jnp.float32) for _ in range(NQ)]
            for l in range(L):
                wgt = wv[l // LANES][l % LANES]
                for q in range(NQ):
                    acc[q] = acc[q] + wgt * rv[
                        base + l, pl.ds(q * LANES, LANES)]
            for q in range(NQ):
                c_v[bag, pl.ds(q * LANES, LANES)] = jnp.where(
                    acc[q] > 0.0, 1.0, 0.0).astype(jnp.float32)
            return carry

        lax.fori_loop(0, CHUNK_BAGS, bag_body, 0)
        pltpu.sync_copy(
            c_v, c_h.at[pl.ds(wid * BAGS_PER_W + ch * CHUNK_BAGS, CHUNK_BAGS)])

    fire_chunk(0, 0)
    for ch in range(NCHUNK):
        par = ch % 2
        if ch + 1 < NCHUNK:
            fire_chunk(ch + 1, 1 - par)
        drain_chunk(par)
        compute_chunk(ch, par)


def _sparse_stage(emb, idx_arr, w_arr):
    mesh = plsc.VectorSubcoreMesh(core_axis_name="c", subcore_axis_name="s")
    fn = pl.kernel(
        _sc_body,
        out_type=jax.ShapeDtypeStruct((BAGS, C), jnp.float32),
        mesh=mesh,
        scratch_types=[
            pltpu.VMEM((CHUNK_IDX,), jnp.int32),
            pltpu.VMEM((CHUNK_BAGS, WPAD), jnp.float32),
            pltpu.VMEM((CHUNK_IDX, C), jnp.float32),
            pltpu.VMEM((CHUNK_IDX, C), jnp.float32),
            pltpu.VMEM((CHUNK_BAGS, C), jnp.float32),
            pltpu.SemaphoreType.DMA,
            pltpu.SemaphoreType.DMA,
        ],
    )
    return fn(emb, idx_arr, w_arr)


def _dec_body(c_ref, wt_ref, b_ref, o_ref):
    o_ref[...] = jnp.dot(
        c_ref[...], wt_ref[...],
        preferred_element_type=jnp.float32,
        precision=lax.Precision.HIGHEST,
    ) + b_ref[...]


def _decoder_stage(c, wt, b2):
    bm = 256
    return pl.pallas_call(
        _dec_body,
        grid=(BAGS // bm,),
        in_specs=[
            pl.BlockSpec((bm, C), lambda i: (i, 0)),
            pl.BlockSpec((C, NCLS), lambda i: (0, 0)),
            pl.BlockSpec((1, NCLS), lambda i: (0, 0)),
        ],
        out_specs=pl.BlockSpec((bm, NCLS), lambda i: (i, 0)),
        out_shape=jax.ShapeDtypeStruct((BAGS, NCLS), jnp.float32),
    )(c, wt, b2)


def kernel(v, emb, W_dec, b_dec):
    keys = v[:, :, 0, :2]
    vals = v[:, :, 1, :2]
    idx = jnp.transpose(keys, (2, 0, 1)).reshape(BAGS, L).astype(jnp.int32)
    wts = jnp.transpose(vals, (2, 0, 1)).reshape(BAGS, L)

    idx_arr = idx.reshape(NW, NCHUNK, CHUNK_IDX)
    w_arr = jnp.pad(wts, ((0, 0), (0, WPAD - L))).reshape(
        NW, NCHUNK, CHUNK_BAGS, WPAD)

    c = _sparse_stage(emb, idx_arr, w_arr)
    out = _decoder_stage(c, W_dec.T, b_dec.reshape(1, NCLS))
    o1 = out[:B]
    o2 = out[B:]
    return (o1, o2, o2)
